# baseline (speedup 1.0000x reference)
import functools
import jax
import jax.numpy as jnp
from jax.experimental import pallas as pl

KNN = 32


def _lrelu(x):
    return jnp.where(x >= 0, x, 0.2 * x)


def _d2_kernel(xt_ref, d2_ref):
    xt = xt_ref[0]
    sq = jnp.sum(xt * xt, axis=-1)
    g = jax.lax.dot_general(xt, xt, (((1,), (1,)), ((), ())),
                            preferred_element_type=jnp.float32)
    d2_ref[0] = sq[:, None] + sq[None, :] - 2.0 * g


def _pallas_d2(xt):
    B, N, C = xt.shape
    return pl.pallas_call(
        _d2_kernel,
        grid=(B,),
        in_specs=[pl.BlockSpec((1, N, C), lambda b: (b, 0, 0))],
        out_specs=pl.BlockSpec((1, N, N), lambda b: (b, 0, 0)),
        out_shape=jax.ShapeDtypeStruct((B, N, N), jnp.float32),
    )(xt)


def _z_kernel(w_ref, f_ref, z_ref):
    z_ref[0] = jnp.dot(w_ref[...], f_ref[0],
                       preferred_element_type=jnp.float32)


def _pallas_z(W, f):
    B, C2, M = f.shape
    O = W.shape[0]
    CHUNK = 2048
    return pl.pallas_call(
        _z_kernel,
        grid=(B, M // CHUNK),
        in_specs=[
            pl.BlockSpec((O, C2), lambda b, m: (0, 0)),
            pl.BlockSpec((1, C2, CHUNK), lambda b, m: (b, 0, m)),
        ],
        out_specs=pl.BlockSpec((1, O, CHUNK), lambda b, m: (b, 0, m)),
        out_shape=jax.ShapeDtypeStruct((B, O, M), jnp.float32),
    )(W, f)


def _edge_layer(h, W):
    B, C, N = h.shape
    xt = jnp.transpose(h, (0, 2, 1))
    d2 = _pallas_d2(xt)
    dist = jnp.sqrt(jnp.clip(d2, 1e-12, None))
    _, idx = jax.lax.top_k(-dist, KNN + 1)
    idx = idx[:, :, 1:]

    xt2 = xt.reshape(B * N, C)
    gidx = (idx + (jnp.arange(B) * N)[:, None, None]).reshape(-1)
    feat = xt2[gidx].reshape(B, N, KNN, C)
    xc = xt2.reshape(B, N, 1, C)
    f = jnp.concatenate(
        [feat - xc, jnp.broadcast_to(xc, (B, N, KNN, C))], axis=3)
    f = jnp.transpose(f, (0, 3, 1, 2))
    z = _pallas_z(W, f.reshape(B, 2 * C, N * KNN)).reshape(
        B, W.shape[0], N, KNN)
    cnt = N * KNN
    mean = jnp.sum(z, axis=(2, 3)) / cnt
    ez2 = jnp.sum(z * z, axis=(2, 3)) / cnt
    var = ez2 - mean * mean
    zmax = jnp.max(z, axis=3)
    hn = _lrelu((zmax - mean[:, :, None]) / jnp.sqrt(var[:, :, None] + 1e-5))
    return hn


def _tail_kernel(xcat_ref, wh_ref, fc1w_ref, fc1b_ref, fc2w_ref, fc2b_ref,
                 out_ref):
    xcat = xcat_ref[0]
    z = jnp.dot(wh_ref[...], xcat, preferred_element_type=jnp.float32)
    m = jnp.mean(z, axis=1, keepdims=True)
    v = jnp.mean((z - m) ** 2, axis=1, keepdims=True)
    zn = _lrelu((z - m) / jnp.sqrt(v + 1e-5))
    g = jnp.max(zn, axis=1, keepdims=True)
    y1 = jnp.dot(fc1w_ref[...], g, preferred_element_type=jnp.float32)
    y1 = y1 + fc1b_ref[...].reshape(-1, 1)
    m1 = jnp.mean(y1)
    v1 = jnp.mean((y1 - m1) ** 2)
    y1 = _lrelu((y1 - m1) / jnp.sqrt(v1 + 1e-5))
    y2 = jnp.dot(fc2w_ref[...], y1, preferred_element_type=jnp.float32)
    y2 = y2 + fc2b_ref[...].reshape(-1, 1)
    out_ref[...] = y2.reshape(1, 1, -1)


def kernel(x, n, W0, W1, W2, W3, Wh, fc1_w, fc1_b, fc2_w, fc2_b):
    del n
    B, N, _ = x.shape
    h = jnp.transpose(x, (0, 2, 1))
    res = []
    for W in (W0, W1, W2, W3):
        h = _edge_layer(h, W)
        res.append(h)
    xcat = jnp.concatenate(res, axis=1)

    E = Wh.shape[0]
    tail = pl.pallas_call(
        _tail_kernel,
        grid=(B,),
        in_specs=[
            pl.BlockSpec((1, xcat.shape[1], N), lambda b: (b, 0, 0)),
            pl.BlockSpec(Wh.shape, lambda b: (0, 0)),
            pl.BlockSpec(fc1_w.shape, lambda b: (0, 0)),
            pl.BlockSpec(fc1_b.shape, lambda b: (0,)),
            pl.BlockSpec(fc2_w.shape, lambda b: (0, 0)),
            pl.BlockSpec(fc2_b.shape, lambda b: (0,)),
        ],
        out_specs=pl.BlockSpec((1, 1, E), lambda b: (b, 0, 0)),
        out_shape=jax.ShapeDtypeStruct((B, 1, E), jnp.float32),
    )
    return tail(xcat, Wh, fc1_w, fc1_b, fc2_w, fc2_b).reshape(B, E)


# speedup vs baseline: 1.0749x; 1.0749x over previous
import functools
import jax
import jax.numpy as jnp
from jax.experimental import pallas as pl

KNN = 32


def _lrelu(x):
    return jnp.where(x >= 0, x, 0.2 * x)


def _d2_kernel(xt_ref, d2_ref):
    xt = xt_ref[0]
    sq = jnp.sum(xt * xt, axis=-1)
    g = jax.lax.dot_general(xt, xt, (((1,), (1,)), ((), ())),
                            preferred_element_type=jnp.float32)
    d2_ref[0] = sq[:, None] + sq[None, :] - 2.0 * g


def _pallas_d2(xt):
    B, N, C = xt.shape
    return pl.pallas_call(
        _d2_kernel,
        grid=(B,),
        in_specs=[pl.BlockSpec((1, N, C), lambda b: (b, 0, 0))],
        out_specs=pl.BlockSpec((1, N, N), lambda b: (b, 0, 0)),
        out_shape=jax.ShapeDtypeStruct((B, N, N), jnp.float32),
    )(xt)


def _edge_kernel(wa_ref, wb_ref, feat_ref, h_ref, zmax_ref, s1_ref, s2_ref):
    NC = 128
    feat = feat_ref[0]
    hb = h_ref[0]
    xc = jnp.repeat(jnp.transpose(hb), KNN, axis=0)
    zn = jax.lax.dot_general(
        wa_ref[...], feat - xc, (((1,), (1,)), ((), ())),
        preferred_element_type=jnp.float32)
    zc = jax.lax.dot_general(
        wb_ref[...], hb, (((1,), (0,)), ((), ())),
        preferred_element_type=jnp.float32)
    z = zn + jnp.repeat(zc, KNN, axis=1)
    O = z.shape[0]
    zmax_ref[0] = jnp.max(z.reshape(O, NC, KNN), axis=2)
    s1_ref[0, 0, 0] = jnp.sum(z, axis=1)
    s2_ref[0, 0, 0] = jnp.sum(z * z, axis=1)


def _pallas_edge(W, feat, h):
    B, C, N = h.shape
    O = W.shape[0]
    NC = 128
    G = N // NC
    Wa = W[:, :C]
    Wb = W[:, C:]
    return pl.pallas_call(
        _edge_kernel,
        grid=(B, G),
        in_specs=[
            pl.BlockSpec((O, C), lambda b, g: (0, 0)),
            pl.BlockSpec((O, C), lambda b, g: (0, 0)),
            pl.BlockSpec((1, NC * KNN, C), lambda b, g: (b, g, 0)),
            pl.BlockSpec((1, C, NC), lambda b, g: (b, 0, g)),
        ],
        out_specs=[
            pl.BlockSpec((1, O, NC), lambda b, g: (b, 0, g)),
            pl.BlockSpec((1, 1, 1, O), lambda b, g: (b, g, 0, 0)),
            pl.BlockSpec((1, 1, 1, O), lambda b, g: (b, g, 0, 0)),
        ],
        out_shape=[
            jax.ShapeDtypeStruct((B, O, N), jnp.float32),
            jax.ShapeDtypeStruct((B, G, 1, O), jnp.float32),
            jax.ShapeDtypeStruct((B, G, 1, O), jnp.float32),
        ],
    )(Wa, Wb, feat, h)


def _edge_layer(h, W):
    B, C, N = h.shape
    xt = jnp.transpose(h, (0, 2, 1))
    d2 = _pallas_d2(xt)
    dist = jnp.sqrt(jnp.clip(d2, 1e-12, None))
    _, idx = jax.lax.top_k(-dist, KNN + 1)
    idx = idx[:, :, 1:]

    xt2 = xt.reshape(B * N, C)
    gidx = (idx + (jnp.arange(B) * N)[:, None, None]).reshape(-1)
    feat = xt2[gidx].reshape(B, N * KNN, C)
    zmax, s1, s2 = _pallas_edge(W, feat, h)
    cnt = N * KNN
    mean = jnp.sum(s1, axis=(1, 2)) / cnt
    ez2 = jnp.sum(s2, axis=(1, 2)) / cnt
    var = ez2 - mean * mean
    hn = _lrelu((zmax - mean[:, :, None]) / jnp.sqrt(var[:, :, None] + 1e-5))
    return hn


def _tail_kernel(xcat_ref, wh_ref, fc1w_ref, fc1b_ref, fc2w_ref, fc2b_ref,
                 out_ref):
    xcat = xcat_ref[0]
    z = jnp.dot(wh_ref[...], xcat, preferred_element_type=jnp.float32)
    m = jnp.mean(z, axis=1, keepdims=True)
    v = jnp.mean((z - m) ** 2, axis=1, keepdims=True)
    zn = _lrelu((z - m) / jnp.sqrt(v + 1e-5))
    g = jnp.max(zn, axis=1, keepdims=True)
    y1 = jnp.dot(fc1w_ref[...], g, preferred_element_type=jnp.float32)
    y1 = y1 + fc1b_ref[...].reshape(-1, 1)
    m1 = jnp.mean(y1)
    v1 = jnp.mean((y1 - m1) ** 2)
    y1 = _lrelu((y1 - m1) / jnp.sqrt(v1 + 1e-5))
    y2 = jnp.dot(fc2w_ref[...], y1, preferred_element_type=jnp.float32)
    y2 = y2 + fc2b_ref[...].reshape(-1, 1)
    out_ref[...] = y2.reshape(1, 1, -1)


def kernel(x, n, W0, W1, W2, W3, Wh, fc1_w, fc1_b, fc2_w, fc2_b):
    del n
    B, N, _ = x.shape
    h = jnp.transpose(x, (0, 2, 1))
    res = []
    for W in (W0, W1, W2, W3):
        h = _edge_layer(h, W)
        res.append(h)
    xcat = jnp.concatenate(res, axis=1)

    E = Wh.shape[0]
    tail = pl.pallas_call(
        _tail_kernel,
        grid=(B,),
        in_specs=[
            pl.BlockSpec((1, xcat.shape[1], N), lambda b: (b, 0, 0)),
            pl.BlockSpec(Wh.shape, lambda b: (0, 0)),
            pl.BlockSpec(fc1_w.shape, lambda b: (0, 0)),
            pl.BlockSpec(fc1_b.shape, lambda b: (0,)),
            pl.BlockSpec(fc2_w.shape, lambda b: (0, 0)),
            pl.BlockSpec(fc2_b.shape, lambda b: (0,)),
        ],
        out_specs=pl.BlockSpec((1, 1, E), lambda b: (b, 0, 0)),
        out_shape=jax.ShapeDtypeStruct((B, 1, E), jnp.float32),
    )
    return tail(xcat, Wh, fc1_w, fc1_b, fc2_w, fc2_b).reshape(B, E)


# speedup vs baseline: 1.1490x; 1.0690x over previous
import functools
import jax
import jax.numpy as jnp
from jax import lax
from jax.experimental import pallas as pl
from jax.experimental.pallas import tpu as pltpu
from jax.experimental.pallas import tpu_sc as plsc

KNN = 32
_BIG = 3.0e38

def _sc_gather_kernel(tab_hbm, gidx_hbm, feat_hbm, idxv, rows, sem):
    step = 128
    nstep = (128 * 1024) // (32 * step)
    wid = lax.axis_index("c") * 16 + lax.axis_index("s")
    base = wid * step * nstep

    def body(j, _c):
        o = base + j * step
        pltpu.sync_copy(gidx_hbm.at[pl.ds(o, step)], idxv)
        pltpu.async_copy(tab_hbm.at[idxv], rows, sem).wait()
        pltpu.sync_copy(rows, feat_hbm.at[pl.ds(o, step)])
        return 0

    lax.fori_loop(0, nstep, body, 0)


def _sc_gather(tab, gidx):
    BN, C = tab.shape
    M = gidx.shape[0]
    run = functools.partial(
        pl.kernel,
        out_type=jax.ShapeDtypeStruct((M, C), jnp.float32),
        mesh=plsc.VectorSubcoreMesh(core_axis_name="c", subcore_axis_name="s"),
        scratch_types=[
            pltpu.VMEM((128,), jnp.int32),
            pltpu.VMEM((128, C), jnp.float32),
            pltpu.SemaphoreType.DMA,
        ],
    )(_sc_gather_kernel)
    return run(tab, gidx)


def _lrelu(x):
    return jnp.where(x >= 0, x, 0.2 * x)


def _d2_kernel(xt_ref, d2_ref):
    xt = xt_ref[0]
    sq = jnp.sum(xt * xt, axis=-1)
    g = jax.lax.dot_general(xt, xt, (((1,), (1,)), ((), ())),
                            preferred_element_type=jnp.float32)
    d2_ref[0] = sq[:, None] + sq[None, :] - 2.0 * g


def _pallas_d2(xt):
    B, N, C = xt.shape
    return pl.pallas_call(
        _d2_kernel,
        grid=(B,),
        in_specs=[pl.BlockSpec((1, N, C), lambda b: (b, 0, 0))],
        out_specs=pl.BlockSpec((1, N, N), lambda b: (b, 0, 0)),
        out_shape=jax.ShapeDtypeStruct((B, N, N), jnp.float32),
    )(xt)


def _edge_kernel(wa_ref, wb_ref, feat_ref, h_ref, zmax_ref, s1_ref, s2_ref):
    NC = 128
    feat = feat_ref[0]
    hb = h_ref[0]
    xc = jnp.repeat(jnp.transpose(hb), KNN, axis=0)
    zn = jax.lax.dot_general(
        wa_ref[...], feat - xc, (((1,), (1,)), ((), ())),
        preferred_element_type=jnp.float32)
    zc = jax.lax.dot_general(
        wb_ref[...], hb, (((1,), (0,)), ((), ())),
        preferred_element_type=jnp.float32)
    z = zn + jnp.repeat(zc, KNN, axis=1)
    O = z.shape[0]
    zmax_ref[0] = jnp.max(z.reshape(O, NC, KNN), axis=2)
    s1_ref[0, 0, 0] = jnp.sum(z, axis=1)
    s2_ref[0, 0, 0] = jnp.sum(z * z, axis=1)


def _pallas_edge(W, feat, h):
    B, C, N = h.shape
    O = W.shape[0]
    NC = 128
    G = N // NC
    Wa = W[:, :C]
    Wb = W[:, C:]
    return pl.pallas_call(
        _edge_kernel,
        grid=(B, G),
        in_specs=[
            pl.BlockSpec((O, C), lambda b, g: (0, 0)),
            pl.BlockSpec((O, C), lambda b, g: (0, 0)),
            pl.BlockSpec((1, NC * KNN, C), lambda b, g: (b, g, 0)),
            pl.BlockSpec((1, C, NC), lambda b, g: (b, 0, g)),
        ],
        out_specs=[
            pl.BlockSpec((1, O, NC), lambda b, g: (b, 0, g)),
            pl.BlockSpec((1, 1, 1, O), lambda b, g: (b, g, 0, 0)),
            pl.BlockSpec((1, 1, 1, O), lambda b, g: (b, g, 0, 0)),
        ],
        out_shape=[
            jax.ShapeDtypeStruct((B, O, N), jnp.float32),
            jax.ShapeDtypeStruct((B, G, 1, O), jnp.float32),
            jax.ShapeDtypeStruct((B, G, 1, O), jnp.float32),
        ],
    )(Wa, Wb, feat, h)


def _edge_layer(h, W):
    B, C, N = h.shape
    xt = jnp.transpose(h, (0, 2, 1))
    d2 = _pallas_d2(xt)
    dist = jnp.sqrt(jnp.clip(d2, 1e-12, None))
    _, idx = jax.lax.top_k(-dist, KNN + 1)
    idx = idx[:, :, 1:]
    gidx = (idx + (jnp.arange(B) * N)[:, None, None]).reshape(-1)

    xt2 = xt.reshape(B * N, C)
    if C % 128 == 0:
        feat = _sc_gather(xt2, gidx).reshape(B, N * KNN, C)
    else:
        feat = xt2[gidx].reshape(B, N * KNN, C)
    zmax, s1, s2 = _pallas_edge(W, feat, h)
    cnt = N * KNN
    mean = jnp.sum(s1, axis=(1, 2)) / cnt
    ez2 = jnp.sum(s2, axis=(1, 2)) / cnt
    var = ez2 - mean * mean
    hn = _lrelu((zmax - mean[:, :, None]) / jnp.sqrt(var[:, :, None] + 1e-5))
    return hn


def _tail_kernel(xcat_ref, wh_ref, fc1w_ref, fc1b_ref, fc2w_ref, fc2b_ref,
                 out_ref):
    xcat = xcat_ref[0]
    z = jnp.dot(wh_ref[...], xcat, preferred_element_type=jnp.float32)
    m = jnp.mean(z, axis=1, keepdims=True)
    v = jnp.mean((z - m) ** 2, axis=1, keepdims=True)
    zn = _lrelu((z - m) / jnp.sqrt(v + 1e-5))
    g = jnp.max(zn, axis=1, keepdims=True)
    y1 = jnp.dot(fc1w_ref[...], g, preferred_element_type=jnp.float32)
    y1 = y1 + fc1b_ref[...].reshape(-1, 1)
    m1 = jnp.mean(y1)
    v1 = jnp.mean((y1 - m1) ** 2)
    y1 = _lrelu((y1 - m1) / jnp.sqrt(v1 + 1e-5))
    y2 = jnp.dot(fc2w_ref[...], y1, preferred_element_type=jnp.float32)
    y2 = y2 + fc2b_ref[...].reshape(-1, 1)
    out_ref[...] = y2.reshape(1, 1, -1)


def kernel(x, n, W0, W1, W2, W3, Wh, fc1_w, fc1_b, fc2_w, fc2_b):
    del n
    B, N, _ = x.shape
    h = jnp.transpose(x, (0, 2, 1))
    res = []
    for W in (W0, W1, W2, W3):
        h = _edge_layer(h, W)
        res.append(h)
    xcat = jnp.concatenate(res, axis=1)

    E = Wh.shape[0]
    tail = pl.pallas_call(
        _tail_kernel,
        grid=(B,),
        in_specs=[
            pl.BlockSpec((1, xcat.shape[1], N), lambda b: (b, 0, 0)),
            pl.BlockSpec(Wh.shape, lambda b: (0, 0)),
            pl.BlockSpec(fc1_w.shape, lambda b: (0, 0)),
            pl.BlockSpec(fc1_b.shape, lambda b: (0,)),
            pl.BlockSpec(fc2_w.shape, lambda b: (0, 0)),
            pl.BlockSpec(fc2_b.shape, lambda b: (0,)),
        ],
        out_specs=pl.BlockSpec((1, 1, E), lambda b: (b, 0, 0)),
        out_shape=jax.ShapeDtypeStruct((B, 1, E), jnp.float32),
    )
    return tail(xcat, Wh, fc1_w, fc1_b, fc2_w, fc2_b).reshape(B, E)


# speedup vs baseline: 1.1550x; 1.0052x over previous
import functools
import jax
import jax.numpy as jnp
from jax import lax
from jax.experimental import pallas as pl
from jax.experimental.pallas import tpu as pltpu
from jax.experimental.pallas import tpu_sc as plsc

KNN = 32
_BIG = 3.0e38

def _sc_gather_kernel(tab_hbm, gidx_hbm, feat_hbm, idxv, rows, sem):
    step = 128
    nstep = (128 * 1024) // (32 * step)
    wid = lax.axis_index("c") * 16 + lax.axis_index("s")
    base = wid * step * nstep

    def body(j, _c):
        o = base + j * step
        pltpu.sync_copy(gidx_hbm.at[pl.ds(o, step)], idxv)
        pltpu.async_copy(tab_hbm.at[idxv], rows, sem).wait()
        pltpu.sync_copy(rows, feat_hbm.at[pl.ds(o, step)])
        return 0

    lax.fori_loop(0, nstep, body, 0)


def _sc_gather(tab, gidx):
    BN, C = tab.shape
    M = gidx.shape[0]
    run = functools.partial(
        pl.kernel,
        out_type=jax.ShapeDtypeStruct((M, C), jnp.float32),
        mesh=plsc.VectorSubcoreMesh(core_axis_name="c", subcore_axis_name="s"),
        scratch_types=[
            pltpu.VMEM((128,), jnp.int32),
            pltpu.VMEM((128, C), jnp.float32),
            pltpu.SemaphoreType.DMA,
        ],
    )(_sc_gather_kernel)
    return run(tab, gidx)


def _lrelu(x):
    return jnp.where(x >= 0, x, 0.2 * x)


def _d2_kernel(xt_ref, d2_ref):
    xt = xt_ref[0]
    sq = jnp.sum(xt * xt, axis=-1)
    g = jax.lax.dot_general(xt, xt, (((1,), (1,)), ((), ())),
                            preferred_element_type=jnp.float32)
    d2 = sq[:, None] + sq[None, :] - 2.0 * g
    d2_ref[0] = -jnp.sqrt(jnp.clip(d2, 1e-12, None))


def _pallas_d2(xt):
    B, N, C = xt.shape
    return pl.pallas_call(
        _d2_kernel,
        grid=(B,),
        in_specs=[pl.BlockSpec((1, N, C), lambda b: (b, 0, 0))],
        out_specs=pl.BlockSpec((1, N, N), lambda b: (b, 0, 0)),
        out_shape=jax.ShapeDtypeStruct((B, N, N), jnp.float32),
    )(xt)


def _edge_kernel(wa_ref, wb_ref, feat_ref, h_ref, zmax_ref, s1_ref, s2_ref):
    NC = 128
    feat = feat_ref[0]
    hb = h_ref[0]
    xc = jnp.repeat(jnp.transpose(hb), KNN, axis=0)
    zn = jax.lax.dot_general(
        wa_ref[...], feat - xc, (((1,), (1,)), ((), ())),
        preferred_element_type=jnp.float32)
    zc = jax.lax.dot_general(
        wb_ref[...], hb, (((1,), (0,)), ((), ())),
        preferred_element_type=jnp.float32)
    z = zn + jnp.repeat(zc, KNN, axis=1)
    O = z.shape[0]
    zmax_ref[0] = jnp.max(z.reshape(O, NC, KNN), axis=2)
    s1_ref[0, 0, 0] = jnp.sum(z, axis=1)
    s2_ref[0, 0, 0] = jnp.sum(z * z, axis=1)


def _pallas_edge(W, feat, h):
    B, C, N = h.shape
    O = W.shape[0]
    NC = 128
    G = N // NC
    Wa = W[:, :C]
    Wb = W[:, C:]
    return pl.pallas_call(
        _edge_kernel,
        grid=(B, G),
        in_specs=[
            pl.BlockSpec((O, C), lambda b, g: (0, 0)),
            pl.BlockSpec((O, C), lambda b, g: (0, 0)),
            pl.BlockSpec((1, NC * KNN, C), lambda b, g: (b, g, 0)),
            pl.BlockSpec((1, C, NC), lambda b, g: (b, 0, g)),
        ],
        out_specs=[
            pl.BlockSpec((1, O, NC), lambda b, g: (b, 0, g)),
            pl.BlockSpec((1, 1, 1, O), lambda b, g: (b, g, 0, 0)),
            pl.BlockSpec((1, 1, 1, O), lambda b, g: (b, g, 0, 0)),
        ],
        out_shape=[
            jax.ShapeDtypeStruct((B, O, N), jnp.float32),
            jax.ShapeDtypeStruct((B, G, 1, O), jnp.float32),
            jax.ShapeDtypeStruct((B, G, 1, O), jnp.float32),
        ],
    )(Wa, Wb, feat, h)


def _edge_layer(h, W):
    B, C, N = h.shape
    xt = jnp.transpose(h, (0, 2, 1))
    ndist = _pallas_d2(xt)
    _, idx = jax.lax.top_k(ndist, KNN + 1)
    idx = idx[:, :, 1:]
    gidx = (idx + (jnp.arange(B) * N)[:, None, None]).reshape(-1)

    xt2 = xt.reshape(B * N, C)
    if C % 128 == 0:
        feat = _sc_gather(xt2, gidx).reshape(B, N * KNN, C)
    else:
        feat = xt2[gidx].reshape(B, N * KNN, C)
    zmax, s1, s2 = _pallas_edge(W, feat, h)
    cnt = N * KNN
    mean = jnp.sum(s1, axis=(1, 2)) / cnt
    ez2 = jnp.sum(s2, axis=(1, 2)) / cnt
    var = ez2 - mean * mean
    hn = _lrelu((zmax - mean[:, :, None]) / jnp.sqrt(var[:, :, None] + 1e-5))
    return hn


def _tail_kernel(xcat_ref, wh_ref, fc1w_ref, fc1b_ref, fc2w_ref, fc2b_ref,
                 out_ref):
    xcat = xcat_ref[0]
    z = jnp.dot(wh_ref[...], xcat, preferred_element_type=jnp.float32)
    m = jnp.mean(z, axis=1, keepdims=True)
    v = jnp.mean((z - m) ** 2, axis=1, keepdims=True)
    zn = _lrelu((z - m) / jnp.sqrt(v + 1e-5))
    g = jnp.max(zn, axis=1, keepdims=True)
    y1 = jnp.dot(fc1w_ref[...], g, preferred_element_type=jnp.float32)
    y1 = y1 + fc1b_ref[...].reshape(-1, 1)
    m1 = jnp.mean(y1)
    v1 = jnp.mean((y1 - m1) ** 2)
    y1 = _lrelu((y1 - m1) / jnp.sqrt(v1 + 1e-5))
    y2 = jnp.dot(fc2w_ref[...], y1, preferred_element_type=jnp.float32)
    y2 = y2 + fc2b_ref[...].reshape(-1, 1)
    out_ref[...] = y2.reshape(1, 1, -1)


def kernel(x, n, W0, W1, W2, W3, Wh, fc1_w, fc1_b, fc2_w, fc2_b):
    del n
    B, N, _ = x.shape
    h = jnp.transpose(x, (0, 2, 1))
    res = []
    for W in (W0, W1, W2, W3):
        h = _edge_layer(h, W)
        res.append(h)
    xcat = jnp.concatenate(res, axis=1)

    E = Wh.shape[0]
    tail = pl.pallas_call(
        _tail_kernel,
        grid=(B,),
        in_specs=[
            pl.BlockSpec((1, xcat.shape[1], N), lambda b: (b, 0, 0)),
            pl.BlockSpec(Wh.shape, lambda b: (0, 0)),
            pl.BlockSpec(fc1_w.shape, lambda b: (0, 0)),
            pl.BlockSpec(fc1_b.shape, lambda b: (0,)),
            pl.BlockSpec(fc2_w.shape, lambda b: (0, 0)),
            pl.BlockSpec(fc2_b.shape, lambda b: (0,)),
        ],
        out_specs=pl.BlockSpec((1, 1, E), lambda b: (b, 0, 0)),
        out_shape=jax.ShapeDtypeStruct((B, 1, E), jnp.float32),
    )
    return tail(xcat, Wh, fc1_w, fc1_b, fc2_w, fc2_b).reshape(B, E)
